# Initial kernel scaffold; baseline (speedup 1.0000x reference)
import jax
import jax.numpy as jnp
from jax.experimental import pallas as pl


def kernel(x, emb_table, W, b_fe, a_l, a_r, W_out, b_out):
    raise NotImplementedError("write your pallas kernel here")


# speedup vs baseline: 120.4454x; 120.4454x over previous
import functools

import jax
import jax.numpy as jnp
from jax import lax
from jax.experimental import pallas as pl
from jax.experimental.pallas import tpu as pltpu
from jax.experimental.pallas import tpu_sc as plsc

N = 1024
B = 16
F = 128
E = 64
K = 32
RT = 128
NW = 32
NPW = (B * N) // NW


def _feat_body(x_ref, w_ref, b_ref, al_ref, ar_ref, emb_ref,
               h_ref, sl_ref, sr_ref):
    xb = x_ref[0]
    h = jnp.dot(xb, w_ref[...], preferred_element_type=jnp.float32)
    h = h + b_ref[0]
    h_ref[0] = h
    emb = emb_ref[...]
    al = al_ref[0]
    ar = ar_ref[0]
    sl = (jnp.sum(h * al[None, :E], axis=1)
          + jnp.sum(emb * al[None, E:], axis=1))
    sr = (jnp.sum(h * ar[None, :E], axis=1)
          + jnp.sum(emb * ar[None, E:], axis=1))
    sl_ref[0] = sl.reshape(8, N // 8)
    sr_ref[0] = sr.reshape(8, N // 8)


_feat_call = pl.pallas_call(
    _feat_body,
    grid=(B,),
    in_specs=[
        pl.BlockSpec((1, N, F), lambda i: (i, 0, 0)),
        pl.BlockSpec((F, E), lambda i: (0, 0)),
        pl.BlockSpec((1, E), lambda i: (0, 0)),
        pl.BlockSpec((1, 2 * E), lambda i: (0, 0)),
        pl.BlockSpec((1, 2 * E), lambda i: (0, 0)),
        pl.BlockSpec((N, E), lambda i: (0, 0)),
    ],
    out_specs=[
        pl.BlockSpec((1, N, E), lambda i: (i, 0, 0)),
        pl.BlockSpec((1, 8, N // 8), lambda i: (i, 0, 0)),
        pl.BlockSpec((1, 8, N // 8), lambda i: (i, 0, 0)),
    ],
    out_shape=[
        jax.ShapeDtypeStruct((B, N, E), jnp.float32),
        jax.ShapeDtypeStruct((B, 8, N // 8), jnp.float32),
        jax.ShapeDtypeStruct((B, 8, N // 8), jnp.float32),
    ],
)


def _topk_body(embb_ref, emb_ref, idx_ref):
    eb = embb_ref[...]
    e = emb_ref[...]
    dot = lax.dot_general(eb, e, (((1,), (1,)), ((), ())),
                          preferred_element_type=jnp.float32)
    nb = jnp.sqrt(jnp.sum(eb * eb, axis=1))
    nf = jnp.sqrt(jnp.sum(e * e, axis=1))
    sim = dot / (nb[:, None] * nf[None, :])
    colio = lax.broadcasted_iota(jnp.int32, (RT, N), 1)
    kio = lax.broadcasted_iota(jnp.int32, (RT, K), 1)

    def body(k, carry):
        s, idxs = carry
        m = jnp.max(s, axis=1, keepdims=True)
        am = jnp.min(jnp.where(s >= m, colio, N), axis=1, keepdims=True)
        idxs = jnp.where(kio == k, am, idxs)
        s = jnp.where(colio == am, -jnp.inf, s)
        return s, idxs

    _, idxs = lax.fori_loop(0, K, body,
                            (sim, jnp.zeros((RT, K), jnp.int32)))
    idx_ref[...] = idxs


_topk_call = pl.pallas_call(
    _topk_body,
    grid=(N // RT,),
    in_specs=[
        pl.BlockSpec((RT, E), lambda i: (i, 0)),
        pl.BlockSpec((N, E), lambda i: (0, 0)),
    ],
    out_specs=pl.BlockSpec((RT, K), lambda i: (i, 0)),
    out_shape=jax.ShapeDtypeStruct((N, K), jnp.int32),
)


def _sc_edge_body(h_hbm, sl_hbm, sr_hbm, idx_hbm, wout_hbm, y_hbm,
                  h_v, sr_v, sl_v, idx_v, wout_v, y_v):
    cid = lax.axis_index("c")
    sid = lax.axis_index("s")
    w = sid * 2 + cid
    b = w // 2
    half = w % 2
    nbase = half * NPW

    pltpu.sync_copy(h_hbm.at[pl.ds(b * N * E, N * E)], h_v)
    pltpu.sync_copy(sr_hbm.at[pl.ds(b * N, N)], sr_v)
    pltpu.sync_copy(sl_hbm.at[pl.ds(b * N + nbase, NPW)], sl_v)
    pltpu.sync_copy(idx_hbm.at[pl.ds(nbase * K, NPW * K)], idx_v)
    pltpu.sync_copy(wout_hbm, wout_v)

    lanei = lax.iota(jnp.int32, 16)
    lane0 = lanei == 0
    w0 = wout_v[0:16]
    w1 = wout_v[16:32]
    w2 = wout_v[32:48]
    w3 = wout_v[48:64]

    def node_body(n, carry):
        iv0 = idx_v[pl.ds(n * K, 16)]
        iv1 = idx_v[pl.ds(n * K + 16, 16)]
        nvec = jnp.full((16,), n, jnp.int32)
        s0 = plsc.load_gather(sr_v, [iv0])
        s1 = plsc.load_gather(sr_v, [iv1])
        slv = plsc.load_gather(sl_v, [nvec])
        l0 = slv + s0
        l1 = slv + s1
        l0 = jnp.where(l0 >= 0, l0, 0.2 * l0)
        l1 = jnp.where(l1 >= 0, l1, 0.2 * l1)
        m = jnp.max(jnp.maximum(l0, l1))
        e0 = jnp.exp(l0 - m)
        e1 = jnp.exp(l1 - m)
        denom = jnp.full((16,), jnp.sum(e0) + jnp.sum(e1), jnp.float32)
        a0 = e0 / denom
        a1 = e1 / denom
        ad0 = iv0 * E
        ad1 = iv1 * E
        acc0 = jnp.zeros((16,), jnp.float32)
        acc1 = jnp.zeros((16,), jnp.float32)
        acc2 = jnp.zeros((16,), jnp.float32)
        acc3 = jnp.zeros((16,), jnp.float32)
        for k in range(16):
            ak = a0[k]
            base = ad0[k]
            acc0 = acc0 + ak * plsc.load_gather(h_v, [base + lanei])
            acc1 = acc1 + ak * plsc.load_gather(h_v, [base + 16 + lanei])
            acc2 = acc2 + ak * plsc.load_gather(h_v, [base + 32 + lanei])
            acc3 = acc3 + ak * plsc.load_gather(h_v, [base + 48 + lanei])
        for k in range(16):
            ak = a1[k]
            base = ad1[k]
            acc0 = acc0 + ak * plsc.load_gather(h_v, [base + lanei])
            acc1 = acc1 + ak * plsc.load_gather(h_v, [base + 16 + lanei])
            acc2 = acc2 + ak * plsc.load_gather(h_v, [base + 32 + lanei])
            acc3 = acc3 + ak * plsc.load_gather(h_v, [base + 48 + lanei])
        r = (jnp.maximum(acc0, 0.0) * w0
             + jnp.maximum(acc1, 0.0) * w1
             + jnp.maximum(acc2, 0.0) * w2
             + jnp.maximum(acc3, 0.0) * w3)
        y = jnp.sum(r)
        plsc.store_scatter(y_v, [nvec], jnp.full((16,), y, jnp.float32),
                           mask=lane0)
        return carry

    lax.fori_loop(0, NPW, node_body, 0)
    pltpu.sync_copy(y_v, y_hbm.at[pl.ds(b * N + nbase, NPW)])


@functools.cache
def _get_sc_edge():
    mesh = plsc.VectorSubcoreMesh(core_axis_name="c", subcore_axis_name="s")
    return pl.kernel(
        _sc_edge_body,
        out_type=jax.ShapeDtypeStruct((B * N,), jnp.float32),
        mesh=mesh,
        compiler_params=pltpu.CompilerParams(needs_layout_passes=False),
        scratch_types=[
            pltpu.VMEM((N * E,), jnp.float32),
            pltpu.VMEM((N,), jnp.float32),
            pltpu.VMEM((NPW,), jnp.float32),
            pltpu.VMEM((NPW * K,), jnp.int32),
            pltpu.VMEM((E,), jnp.float32),
            pltpu.VMEM((NPW,), jnp.float32),
        ],
    )


def kernel(x, emb_table, W, b_fe, a_l, a_r, W_out, b_out):
    h, sl, sr = _feat_call(x, W, b_fe.reshape(1, E),
                           a_l.reshape(1, 2 * E), a_r.reshape(1, 2 * E),
                           emb_table)
    idx = _topk_call(emb_table, emb_table)
    y = _get_sc_edge()(h.reshape(-1), sl.reshape(-1), sr.reshape(-1),
                       idx.reshape(-1), W_out.reshape(-1))
    return y.reshape(B, N) + b_out[0]


# speedup vs baseline: 126.4101x; 1.0495x over previous
import functools

import jax
import jax.numpy as jnp
from jax import lax
from jax.experimental import pallas as pl
from jax.experimental.pallas import tpu as pltpu
from jax.experimental.pallas import tpu_sc as plsc

N = 1024
B = 16
F = 128
E = 64
K = 32
RT = 128
NW = 32
NPW = (B * N) // NW


def _feat_body(x_ref, w_ref, b_ref, al_ref, ar_ref, emb_ref,
               h_ref, sl_ref, sr_ref):
    xb = x_ref[0]
    h = jnp.dot(xb, w_ref[...], preferred_element_type=jnp.float32)
    h = h + b_ref[0]
    h_ref[0] = h
    emb = emb_ref[...]
    al = al_ref[0]
    ar = ar_ref[0]
    sl = (jnp.sum(h * al[None, :E], axis=1)
          + jnp.sum(emb * al[None, E:], axis=1))
    sr = (jnp.sum(h * ar[None, :E], axis=1)
          + jnp.sum(emb * ar[None, E:], axis=1))
    sl_ref[0] = sl.reshape(8, N // 8)
    sr_ref[0] = sr.reshape(8, N // 8)


_feat_call = pl.pallas_call(
    _feat_body,
    grid=(B,),
    in_specs=[
        pl.BlockSpec((1, N, F), lambda i: (i, 0, 0)),
        pl.BlockSpec((F, E), lambda i: (0, 0)),
        pl.BlockSpec((1, E), lambda i: (0, 0)),
        pl.BlockSpec((1, 2 * E), lambda i: (0, 0)),
        pl.BlockSpec((1, 2 * E), lambda i: (0, 0)),
        pl.BlockSpec((N, E), lambda i: (0, 0)),
    ],
    out_specs=[
        pl.BlockSpec((1, N, E), lambda i: (i, 0, 0)),
        pl.BlockSpec((1, 8, N // 8), lambda i: (i, 0, 0)),
        pl.BlockSpec((1, 8, N // 8), lambda i: (i, 0, 0)),
    ],
    out_shape=[
        jax.ShapeDtypeStruct((B, N, E), jnp.float32),
        jax.ShapeDtypeStruct((B, 8, N // 8), jnp.float32),
        jax.ShapeDtypeStruct((B, 8, N // 8), jnp.float32),
    ],
)


def _topk_body(embb_ref, emb_ref, idx_ref):
    eb = embb_ref[...]
    e = emb_ref[...]
    nf = jax.lax.rsqrt(jnp.sum(e * e, axis=1))
    ehat = e * nf[:, None]
    s0 = lax.dot_general(ehat, eb, (((1,), (1,)), ((), ())),
                         preferred_element_type=jnp.float32)
    rowio = lax.broadcasted_iota(jnp.int32, (N, RT), 0)
    kio = lax.broadcasted_iota(jnp.int32, (K, RT), 0)

    def body(k, carry):
        s, idxs = carry
        m = jnp.max(s, axis=0, keepdims=True)
        am = jnp.min(jnp.where(s >= m, rowio, N), axis=0, keepdims=True)
        idxs = jnp.where(kio == k, am, idxs)
        s = jnp.where(rowio == am, -jnp.inf, s)
        return s, idxs

    _, idxs = lax.fori_loop(0, K, body,
                            (s0, jnp.zeros((K, RT), jnp.int32)))
    idx_ref[...] = idxs.T


_topk_call = pl.pallas_call(
    _topk_body,
    grid=(N // RT,),
    in_specs=[
        pl.BlockSpec((RT, E), lambda i: (i, 0)),
        pl.BlockSpec((N, E), lambda i: (0, 0)),
    ],
    out_specs=pl.BlockSpec((RT, K), lambda i: (i, 0)),
    out_shape=jax.ShapeDtypeStruct((N, K), jnp.int32),
)


def _sc_edge_body(h_hbm, sl_hbm, sr_hbm, idx_hbm, wout_hbm, y_hbm,
                  h_v, sr_v, sl_v, idx_v, wout_v, y_v):
    cid = lax.axis_index("c")
    sid = lax.axis_index("s")
    w = sid * 2 + cid
    b = w // 2
    half = w % 2
    nbase = half * NPW

    pltpu.sync_copy(h_hbm.at[pl.ds(b * N * E, N * E)], h_v)
    pltpu.sync_copy(sr_hbm.at[pl.ds(b * N, N)], sr_v)
    pltpu.sync_copy(sl_hbm.at[pl.ds(b * N + nbase, NPW)], sl_v)
    pltpu.sync_copy(idx_hbm.at[pl.ds(nbase * K, NPW * K)], idx_v)
    pltpu.sync_copy(wout_hbm, wout_v)

    lanei = lax.iota(jnp.int32, 16)
    lane0 = lanei == 0
    w0 = wout_v[0:16]
    w1 = wout_v[16:32]
    w2 = wout_v[32:48]
    w3 = wout_v[48:64]

    def node_body(n, carry):
        iv0 = idx_v[pl.ds(n * K, 16)]
        iv1 = idx_v[pl.ds(n * K + 16, 16)]
        nvec = jnp.full((16,), n, jnp.int32)
        s0 = plsc.load_gather(sr_v, [iv0])
        s1 = plsc.load_gather(sr_v, [iv1])
        slv = plsc.load_gather(sl_v, [nvec])
        l0 = slv + s0
        l1 = slv + s1
        l0 = jnp.where(l0 >= 0, l0, 0.2 * l0)
        l1 = jnp.where(l1 >= 0, l1, 0.2 * l1)
        m = jnp.max(jnp.maximum(l0, l1))
        e0 = jnp.exp(l0 - m)
        e1 = jnp.exp(l1 - m)
        denom = jnp.full((16,), jnp.sum(e0) + jnp.sum(e1), jnp.float32)
        a0 = e0 / denom
        a1 = e1 / denom
        ad0 = iv0 * E
        ad1 = iv1 * E
        acc0 = jnp.zeros((16,), jnp.float32)
        acc1 = jnp.zeros((16,), jnp.float32)
        acc2 = jnp.zeros((16,), jnp.float32)
        acc3 = jnp.zeros((16,), jnp.float32)
        for k in range(16):
            ak = a0[k]
            base = ad0[k]
            acc0 = acc0 + ak * plsc.load_gather(h_v, [base + lanei])
            acc1 = acc1 + ak * plsc.load_gather(h_v, [base + 16 + lanei])
            acc2 = acc2 + ak * plsc.load_gather(h_v, [base + 32 + lanei])
            acc3 = acc3 + ak * plsc.load_gather(h_v, [base + 48 + lanei])
        for k in range(16):
            ak = a1[k]
            base = ad1[k]
            acc0 = acc0 + ak * plsc.load_gather(h_v, [base + lanei])
            acc1 = acc1 + ak * plsc.load_gather(h_v, [base + 16 + lanei])
            acc2 = acc2 + ak * plsc.load_gather(h_v, [base + 32 + lanei])
            acc3 = acc3 + ak * plsc.load_gather(h_v, [base + 48 + lanei])
        r = (jnp.maximum(acc0, 0.0) * w0
             + jnp.maximum(acc1, 0.0) * w1
             + jnp.maximum(acc2, 0.0) * w2
             + jnp.maximum(acc3, 0.0) * w3)
        y = jnp.sum(r)
        plsc.store_scatter(y_v, [nvec], jnp.full((16,), y, jnp.float32),
                           mask=lane0)
        return carry

    lax.fori_loop(0, NPW, node_body, 0)
    pltpu.sync_copy(y_v, y_hbm.at[pl.ds(b * N + nbase, NPW)])


@functools.cache
def _get_sc_edge():
    mesh = plsc.VectorSubcoreMesh(core_axis_name="c", subcore_axis_name="s")
    return pl.kernel(
        _sc_edge_body,
        out_type=jax.ShapeDtypeStruct((B * N,), jnp.float32),
        mesh=mesh,
        compiler_params=pltpu.CompilerParams(needs_layout_passes=False),
        scratch_types=[
            pltpu.VMEM((N * E,), jnp.float32),
            pltpu.VMEM((N,), jnp.float32),
            pltpu.VMEM((NPW,), jnp.float32),
            pltpu.VMEM((NPW * K,), jnp.int32),
            pltpu.VMEM((E,), jnp.float32),
            pltpu.VMEM((NPW,), jnp.float32),
        ],
    )


def kernel(x, emb_table, W, b_fe, a_l, a_r, W_out, b_out):
    h, sl, sr = _feat_call(x, W, b_fe.reshape(1, E),
                           a_l.reshape(1, 2 * E), a_r.reshape(1, 2 * E),
                           emb_table)
    idx = _topk_call(emb_table, emb_table)
    y = _get_sc_edge()(h.reshape(-1), sl.reshape(-1), sr.reshape(-1),
                       idx.reshape(-1), W_out.reshape(-1))
    return y.reshape(B, N) + b_out[0]


# speedup vs baseline: 127.3890x; 1.0077x over previous
import functools

import jax
import jax.numpy as jnp
from jax import lax
from jax.experimental import pallas as pl
from jax.experimental.pallas import tpu as pltpu
from jax.experimental.pallas import tpu_sc as plsc

N = 1024
B = 16
F = 128
E = 64
K = 32
RT = 128
NW = 32
NPW = (B * N) // NW


def _feat_body(x_ref, w_ref, b_ref, al_ref, ar_ref, emb_ref,
               h_ref, sl_ref, sr_ref):
    xb = x_ref[0]
    h = jnp.dot(xb, w_ref[...], preferred_element_type=jnp.float32)
    h = h + b_ref[0]
    h_ref[0] = h
    emb = emb_ref[...]
    al = al_ref[0]
    ar = ar_ref[0]
    sl = (jnp.sum(h * al[None, :E], axis=1)
          + jnp.sum(emb * al[None, E:], axis=1))
    sr = (jnp.sum(h * ar[None, :E], axis=1)
          + jnp.sum(emb * ar[None, E:], axis=1))
    sl_ref[0] = sl.reshape(8, N // 8)
    sr_ref[0] = sr.reshape(8, N // 8)


_feat_call = pl.pallas_call(
    _feat_body,
    grid=(B,),
    in_specs=[
        pl.BlockSpec((1, N, F), lambda i: (i, 0, 0)),
        pl.BlockSpec((F, E), lambda i: (0, 0)),
        pl.BlockSpec((1, E), lambda i: (0, 0)),
        pl.BlockSpec((1, 2 * E), lambda i: (0, 0)),
        pl.BlockSpec((1, 2 * E), lambda i: (0, 0)),
        pl.BlockSpec((N, E), lambda i: (0, 0)),
    ],
    out_specs=[
        pl.BlockSpec((1, N, E), lambda i: (i, 0, 0)),
        pl.BlockSpec((1, 8, N // 8), lambda i: (i, 0, 0)),
        pl.BlockSpec((1, 8, N // 8), lambda i: (i, 0, 0)),
    ],
    out_shape=[
        jax.ShapeDtypeStruct((B, N, E), jnp.float32),
        jax.ShapeDtypeStruct((B, 8, N // 8), jnp.float32),
        jax.ShapeDtypeStruct((B, 8, N // 8), jnp.float32),
    ],
)


def _topk_body(embb_ref, emb_ref, idx_ref):
    eb = embb_ref[...]
    e = emb_ref[...]
    nf = jnp.sqrt(jnp.sum(e * e, axis=1))
    dot = lax.dot_general(e, eb, (((1,), (1,)), ((), ())),
                          preferred_element_type=jnp.float32)
    s0 = dot / nf[:, None]
    rowio = lax.broadcasted_iota(jnp.int32, (N, RT), 0)
    kio = lax.broadcasted_iota(jnp.int32, (K, RT), 0)

    def body(k, carry):
        s, idxs = carry
        m = jnp.max(s, axis=0, keepdims=True)
        am = jnp.min(jnp.where(s >= m, rowio, N), axis=0, keepdims=True)
        idxs = jnp.where(kio == k, am, idxs)
        s = jnp.where(rowio == am, -jnp.inf, s)
        return s, idxs

    _, idxs = lax.fori_loop(0, K, body,
                            (s0, jnp.zeros((K, RT), jnp.int32)))
    idx_ref[0] = idxs


_topk_call = pl.pallas_call(
    _topk_body,
    grid=(N // RT,),
    in_specs=[
        pl.BlockSpec((RT, E), lambda i: (i, 0)),
        pl.BlockSpec((N, E), lambda i: (0, 0)),
    ],
    out_specs=pl.BlockSpec((1, K, RT), lambda i: (i, 0, 0)),
    out_shape=jax.ShapeDtypeStruct((N // RT, K, RT), jnp.int32),
)


def _sc_edge_body(h_hbm, sl_hbm, sr_hbm, idx_hbm, wout_hbm, y_hbm,
                  h_v, sr_v, sl_v, idx_v, wout_v, y_v):
    cid = lax.axis_index("c")
    sid = lax.axis_index("s")
    w = sid * 2 + cid
    b = w // 2
    half = w % 2
    nbase = half * NPW

    pltpu.sync_copy(h_hbm.at[pl.ds(b * N * E, N * E)], h_v)
    pltpu.sync_copy(sr_hbm.at[pl.ds(b * N, N)], sr_v)
    pltpu.sync_copy(sl_hbm.at[pl.ds(b * N + nbase, NPW)], sl_v)
    pltpu.sync_copy(idx_hbm.at[pl.ds(nbase * K, NPW * K)], idx_v)
    pltpu.sync_copy(wout_hbm, wout_v)

    lanei = lax.iota(jnp.int32, 16)
    lane0 = lanei == 0
    w0 = wout_v[0:16]
    w1 = wout_v[16:32]
    w2 = wout_v[32:48]
    w3 = wout_v[48:64]

    def node_body(n, carry):
        iv0 = idx_v[pl.ds(n * K, 16)]
        iv1 = idx_v[pl.ds(n * K + 16, 16)]
        nvec = jnp.full((16,), n, jnp.int32)
        s0 = plsc.load_gather(sr_v, [iv0])
        s1 = plsc.load_gather(sr_v, [iv1])
        slv = plsc.load_gather(sl_v, [nvec])
        l0 = slv + s0
        l1 = slv + s1
        l0 = jnp.where(l0 >= 0, l0, 0.2 * l0)
        l1 = jnp.where(l1 >= 0, l1, 0.2 * l1)
        m = jnp.max(jnp.maximum(l0, l1))
        e0 = jnp.exp(l0 - m)
        e1 = jnp.exp(l1 - m)
        denom = jnp.full((16,), jnp.sum(e0) + jnp.sum(e1), jnp.float32)
        a0 = e0 / denom
        a1 = e1 / denom
        ad0 = iv0 * E
        ad1 = iv1 * E
        acc0 = jnp.zeros((16,), jnp.float32)
        acc1 = jnp.zeros((16,), jnp.float32)
        acc2 = jnp.zeros((16,), jnp.float32)
        acc3 = jnp.zeros((16,), jnp.float32)
        for k in range(16):
            ak = a0[k]
            base = ad0[k]
            acc0 = acc0 + ak * plsc.load_gather(h_v, [base + lanei])
            acc1 = acc1 + ak * plsc.load_gather(h_v, [base + 16 + lanei])
            acc2 = acc2 + ak * plsc.load_gather(h_v, [base + 32 + lanei])
            acc3 = acc3 + ak * plsc.load_gather(h_v, [base + 48 + lanei])
        for k in range(16):
            ak = a1[k]
            base = ad1[k]
            acc0 = acc0 + ak * plsc.load_gather(h_v, [base + lanei])
            acc1 = acc1 + ak * plsc.load_gather(h_v, [base + 16 + lanei])
            acc2 = acc2 + ak * plsc.load_gather(h_v, [base + 32 + lanei])
            acc3 = acc3 + ak * plsc.load_gather(h_v, [base + 48 + lanei])
        r = (jnp.maximum(acc0, 0.0) * w0
             + jnp.maximum(acc1, 0.0) * w1
             + jnp.maximum(acc2, 0.0) * w2
             + jnp.maximum(acc3, 0.0) * w3)
        y = jnp.sum(r)
        plsc.store_scatter(y_v, [nvec], jnp.full((16,), y, jnp.float32),
                           mask=lane0)
        return carry

    lax.fori_loop(0, NPW, node_body, 0)
    pltpu.sync_copy(y_v, y_hbm.at[pl.ds(b * N + nbase, NPW)])


@functools.cache
def _get_sc_edge():
    mesh = plsc.VectorSubcoreMesh(core_axis_name="c", subcore_axis_name="s")
    return pl.kernel(
        _sc_edge_body,
        out_type=jax.ShapeDtypeStruct((B * N,), jnp.float32),
        mesh=mesh,
        compiler_params=pltpu.CompilerParams(needs_layout_passes=False),
        scratch_types=[
            pltpu.VMEM((N * E,), jnp.float32),
            pltpu.VMEM((N,), jnp.float32),
            pltpu.VMEM((NPW,), jnp.float32),
            pltpu.VMEM((NPW * K,), jnp.int32),
            pltpu.VMEM((E,), jnp.float32),
            pltpu.VMEM((NPW,), jnp.float32),
        ],
    )


def kernel(x, emb_table, W, b_fe, a_l, a_r, W_out, b_out):
    h, sl, sr = _feat_call(x, W, b_fe.reshape(1, E),
                           a_l.reshape(1, 2 * E), a_r.reshape(1, 2 * E),
                           emb_table)
    idx_t = _topk_call(emb_table, emb_table)
    idx = jnp.transpose(idx_t, (0, 2, 1)).reshape(N, K)
    y = _get_sc_edge()(h.reshape(-1), sl.reshape(-1), sr.reshape(-1),
                       idx.reshape(-1), W_out.reshape(-1))
    return y.reshape(B, N) + b_out[0]


# speedup vs baseline: 129.2014x; 1.0142x over previous
import functools

import jax
import jax.numpy as jnp
from jax import lax
from jax.experimental import pallas as pl
from jax.experimental.pallas import tpu as pltpu
from jax.experimental.pallas import tpu_sc as plsc

N = 1024
B = 16
F = 128
E = 64
K = 32
RT = 128
NW = 32
NPW = (B * N) // NW


def _feat_body(x_ref, w_ref, b_ref, al_ref, ar_ref, emb_ref,
               h_ref, sl_ref, sr_ref):
    xb = x_ref[0]
    h = jnp.dot(xb, w_ref[...], preferred_element_type=jnp.float32)
    h = h + b_ref[0]
    h_ref[0] = h
    emb = emb_ref[...]
    al = al_ref[0]
    ar = ar_ref[0]
    sl = (jnp.sum(h * al[None, :E], axis=1)
          + jnp.sum(emb * al[None, E:], axis=1))
    sr = (jnp.sum(h * ar[None, :E], axis=1)
          + jnp.sum(emb * ar[None, E:], axis=1))
    sl_ref[0] = sl.reshape(8, N // 8)
    sr_ref[0] = sr.reshape(8, N // 8)


_feat_call = pl.pallas_call(
    _feat_body,
    grid=(B,),
    in_specs=[
        pl.BlockSpec((1, N, F), lambda i: (i, 0, 0)),
        pl.BlockSpec((F, E), lambda i: (0, 0)),
        pl.BlockSpec((1, E), lambda i: (0, 0)),
        pl.BlockSpec((1, 2 * E), lambda i: (0, 0)),
        pl.BlockSpec((1, 2 * E), lambda i: (0, 0)),
        pl.BlockSpec((N, E), lambda i: (0, 0)),
    ],
    out_specs=[
        pl.BlockSpec((1, N, E), lambda i: (i, 0, 0)),
        pl.BlockSpec((1, 8, N // 8), lambda i: (i, 0, 0)),
        pl.BlockSpec((1, 8, N // 8), lambda i: (i, 0, 0)),
    ],
    out_shape=[
        jax.ShapeDtypeStruct((B, N, E), jnp.float32),
        jax.ShapeDtypeStruct((B, 8, N // 8), jnp.float32),
        jax.ShapeDtypeStruct((B, 8, N // 8), jnp.float32),
    ],
)


def _topk_body(embb_ref, emb_ref, idx_ref):
    eb = embb_ref[...]
    e = emb_ref[...]
    nf = jnp.sqrt(jnp.sum(e * e, axis=1))
    dot = lax.dot_general(e, eb, (((1,), (1,)), ((), ())),
                          preferred_element_type=jnp.float32)
    s0 = dot / nf[:, None]
    rowio = lax.broadcasted_iota(jnp.int32, (N, RT), 0)
    kio = lax.broadcasted_iota(jnp.int32, (K, RT), 0)

    def body(k, carry):
        s, idxs = carry
        m = jnp.max(s, axis=0, keepdims=True)
        am = jnp.min(jnp.where(s >= m, rowio, N), axis=0, keepdims=True)
        idxs = jnp.where(kio == k, am, idxs)
        s = jnp.where(rowio == am, -jnp.inf, s)
        return s, idxs

    _, idxs = lax.fori_loop(0, K, body,
                            (s0, jnp.zeros((K, RT), jnp.int32)))
    idx_ref[0] = idxs


_topk_call = pl.pallas_call(
    _topk_body,
    grid=(N // RT,),
    in_specs=[
        pl.BlockSpec((RT, E), lambda i: (i, 0)),
        pl.BlockSpec((N, E), lambda i: (0, 0)),
    ],
    out_specs=pl.BlockSpec((1, K, RT), lambda i: (i, 0, 0)),
    out_shape=jax.ShapeDtypeStruct((N // RT, K, RT), jnp.int32),
)


def _sc_edge_body(h_hbm, sl_hbm, sr_hbm, idx_hbm, wout_hbm, y_hbm,
                  h_v, sr_v, sl_v, idx_v, wout_v, y_v):
    cid = lax.axis_index("c")
    sid = lax.axis_index("s")
    w = sid * 2 + cid
    b = w // 2
    half = w % 2
    nbase = half * NPW

    pltpu.sync_copy(h_hbm.at[pl.ds(b * N * E, N * E)], h_v)
    pltpu.sync_copy(sr_hbm.at[pl.ds(b * N, N)], sr_v)
    pltpu.sync_copy(sl_hbm.at[pl.ds(b * N + nbase, NPW)], sl_v)
    pltpu.sync_copy(idx_hbm.at[pl.ds(nbase * K, NPW * K)], idx_v)
    pltpu.sync_copy(wout_hbm, wout_v)

    lanei = lax.iota(jnp.int32, 16)
    lane0 = lanei == 0
    w0 = wout_v[0:16]
    w1 = wout_v[16:32]
    w2 = wout_v[32:48]
    w3 = wout_v[48:64]

    def one_node(n):
        iv0 = idx_v[pl.ds(n * K, 16)]
        iv1 = idx_v[pl.ds(n * K + 16, 16)]
        nvec = jnp.full((16,), n, jnp.int32)
        s0 = plsc.load_gather(sr_v, [iv0])
        s1 = plsc.load_gather(sr_v, [iv1])
        slv = plsc.load_gather(sl_v, [nvec])
        l0 = slv + s0
        l1 = slv + s1
        l0 = jnp.where(l0 >= 0, l0, 0.2 * l0)
        l1 = jnp.where(l1 >= 0, l1, 0.2 * l1)
        e0 = jnp.exp(l0)
        e1 = jnp.exp(l1)
        denom = jnp.full((16,), jnp.sum(e0) + jnp.sum(e1), jnp.float32)
        a0 = e0 / denom
        a1 = e1 / denom
        ad0 = iv0 * E
        ad1 = iv1 * E
        acc0 = jnp.zeros((16,), jnp.float32)
        acc1 = jnp.zeros((16,), jnp.float32)
        acc2 = jnp.zeros((16,), jnp.float32)
        acc3 = jnp.zeros((16,), jnp.float32)
        for av, adv in ((a0, ad0), (a1, ad1)):
            for k in range(16):
                ak = av[k]
                base = adv[k]
                acc0 = acc0 + ak * plsc.load_gather(h_v, [base + lanei])
                acc1 = acc1 + ak * plsc.load_gather(h_v, [base + 16 + lanei])
                acc2 = acc2 + ak * plsc.load_gather(h_v, [base + 32 + lanei])
                acc3 = acc3 + ak * plsc.load_gather(h_v, [base + 48 + lanei])
        r = (jnp.maximum(acc0, 0.0) * w0
             + jnp.maximum(acc1, 0.0) * w1
             + jnp.maximum(acc2, 0.0) * w2
             + jnp.maximum(acc3, 0.0) * w3)
        y = jnp.sum(r)
        plsc.store_scatter(y_v, [nvec], jnp.full((16,), y, jnp.float32),
                           mask=lane0)

    def node_body(i, carry):
        one_node(2 * i)
        one_node(2 * i + 1)
        return carry

    lax.fori_loop(0, NPW // 2, node_body, 0)
    pltpu.sync_copy(y_v, y_hbm.at[pl.ds(b * N + nbase, NPW)])


@functools.cache
def _get_sc_edge():
    mesh = plsc.VectorSubcoreMesh(core_axis_name="c", subcore_axis_name="s")
    return pl.kernel(
        _sc_edge_body,
        out_type=jax.ShapeDtypeStruct((B * N,), jnp.float32),
        mesh=mesh,
        compiler_params=pltpu.CompilerParams(needs_layout_passes=False),
        scratch_types=[
            pltpu.VMEM((N * E,), jnp.float32),
            pltpu.VMEM((N,), jnp.float32),
            pltpu.VMEM((NPW,), jnp.float32),
            pltpu.VMEM((NPW * K,), jnp.int32),
            pltpu.VMEM((E,), jnp.float32),
            pltpu.VMEM((NPW,), jnp.float32),
        ],
    )


def kernel(x, emb_table, W, b_fe, a_l, a_r, W_out, b_out):
    h, sl, sr = _feat_call(x, W, b_fe.reshape(1, E),
                           a_l.reshape(1, 2 * E), a_r.reshape(1, 2 * E),
                           emb_table)
    idx_t = _topk_call(emb_table, emb_table)
    idx = jnp.transpose(idx_t, (0, 2, 1)).reshape(N, K)
    y = _get_sc_edge()(h.reshape(-1), sl.reshape(-1), sr.reshape(-1),
                       idx.reshape(-1), W_out.reshape(-1))
    return y.reshape(B, N) + b_out[0]


# speedup vs baseline: 138.5110x; 1.0721x over previous
import functools

import jax
import jax.numpy as jnp
from jax import lax
from jax.experimental import pallas as pl
from jax.experimental.pallas import tpu as pltpu
from jax.experimental.pallas import tpu_sc as plsc

N = 1024
B = 16
F = 128
E = 64
K = 32
RT = 128
NW = 32
NPW = (B * N) // NW


def _feat_body(x_ref, w_ref, b_ref, al_ref, ar_ref, emb_ref,
               h_ref, sl_ref, sr_ref):
    xb = x_ref[0]
    h = jnp.dot(xb, w_ref[...], preferred_element_type=jnp.float32)
    h = h + b_ref[0]
    h_ref[0] = h
    emb = emb_ref[...]
    al = al_ref[0]
    ar = ar_ref[0]
    sl = (jnp.sum(h * al[None, :E], axis=1)
          + jnp.sum(emb * al[None, E:], axis=1))
    sr = (jnp.sum(h * ar[None, :E], axis=1)
          + jnp.sum(emb * ar[None, E:], axis=1))
    sl_ref[0] = sl.reshape(8, N // 8)
    sr_ref[0] = sr.reshape(8, N // 8)


_feat_call = pl.pallas_call(
    _feat_body,
    grid=(B,),
    in_specs=[
        pl.BlockSpec((1, N, F), lambda i: (i, 0, 0)),
        pl.BlockSpec((F, E), lambda i: (0, 0)),
        pl.BlockSpec((1, E), lambda i: (0, 0)),
        pl.BlockSpec((1, 2 * E), lambda i: (0, 0)),
        pl.BlockSpec((1, 2 * E), lambda i: (0, 0)),
        pl.BlockSpec((N, E), lambda i: (0, 0)),
    ],
    out_specs=[
        pl.BlockSpec((1, N, E), lambda i: (i, 0, 0)),
        pl.BlockSpec((1, 8, N // 8), lambda i: (i, 0, 0)),
        pl.BlockSpec((1, 8, N // 8), lambda i: (i, 0, 0)),
    ],
    out_shape=[
        jax.ShapeDtypeStruct((B, N, E), jnp.float32),
        jax.ShapeDtypeStruct((B, 8, N // 8), jnp.float32),
        jax.ShapeDtypeStruct((B, 8, N // 8), jnp.float32),
    ],
)


def _topk_body(embb_ref, emb_ref, idx_ref):
    eb = embb_ref[...]
    e = emb_ref[...]
    nf = jnp.sqrt(jnp.sum(e * e, axis=1))
    dot = lax.dot_general(e, eb, (((1,), (1,)), ((), ())),
                          preferred_element_type=jnp.float32)
    s0 = dot / nf[:, None]
    rowio = lax.broadcasted_iota(jnp.int32, (N, RT), 0)
    kio = lax.broadcasted_iota(jnp.int32, (K, RT), 0)

    def body(k, carry):
        s, idxs = carry
        m = jnp.max(s, axis=0, keepdims=True)
        am = jnp.min(jnp.where(s >= m, rowio, N), axis=0, keepdims=True)
        idxs = jnp.where(kio == k, am, idxs)
        s = jnp.where(rowio == am, -jnp.inf, s)
        return s, idxs

    _, idxs = lax.fori_loop(0, K, body,
                            (s0, jnp.zeros((K, RT), jnp.int32)))
    idx_ref[0] = idxs


_topk_call = pl.pallas_call(
    _topk_body,
    grid=(N // RT,),
    in_specs=[
        pl.BlockSpec((RT, E), lambda i: (i, 0)),
        pl.BlockSpec((N, E), lambda i: (0, 0)),
    ],
    out_specs=pl.BlockSpec((1, K, RT), lambda i: (i, 0, 0)),
    out_shape=jax.ShapeDtypeStruct((N // RT, K, RT), jnp.int32),
)


def _sc_edge_body(h_hbm, sl_hbm, sr_hbm, idx_hbm, wout_hbm, y_hbm,
                  h_v, sr_v, sl_v, idx_v, wout_v, y_v):
    cid = lax.axis_index("c")
    sid = lax.axis_index("s")
    w = sid * 2 + cid
    b = w // 2
    half = w % 2
    nbase = half * NPW

    pltpu.sync_copy(h_hbm.at[pl.ds(b * N * E, N * E)], h_v)
    pltpu.sync_copy(sr_hbm.at[pl.ds(b * N, N)], sr_v)
    pltpu.sync_copy(sl_hbm.at[pl.ds(b * N + nbase, NPW)], sl_v)
    pltpu.sync_copy(idx_hbm.at[pl.ds(nbase * K, NPW * K)], idx_v)
    pltpu.sync_copy(wout_hbm, wout_v)

    lanei = lax.iota(jnp.int32, 16)
    lane0 = lanei == 0
    w0 = wout_v[0:16]
    w1 = wout_v[16:32]
    w2 = wout_v[32:48]
    w3 = wout_v[48:64]

    def one_node(n):
        iv0 = idx_v[pl.ds(n * K, 16)]
        iv1 = idx_v[pl.ds(n * K + 16, 16)]
        nvec = jnp.full((16,), n, jnp.int32)
        s0 = plsc.load_gather(sr_v, [iv0])
        s1 = plsc.load_gather(sr_v, [iv1])
        slv = plsc.load_gather(sl_v, [nvec])
        l0 = slv + s0
        l1 = slv + s1
        l0 = jnp.where(l0 >= 0, l0, 0.2 * l0)
        l1 = jnp.where(l1 >= 0, l1, 0.2 * l1)
        e0 = jnp.exp(l0)
        e1 = jnp.exp(l1)
        denom = jnp.full((16,), jnp.sum(e0) + jnp.sum(e1), jnp.float32)
        a0 = e0 / denom
        a1 = e1 / denom
        ad0 = iv0 * E
        ad1 = iv1 * E
        acc0 = jnp.zeros((16,), jnp.float32)
        acc1 = jnp.zeros((16,), jnp.float32)
        acc2 = jnp.zeros((16,), jnp.float32)
        acc3 = jnp.zeros((16,), jnp.float32)
        for av, adv in ((a0, ad0), (a1, ad1)):
            for k in range(16):
                ak = av[k]
                base = adv[k]
                acc0 = acc0 + ak * h_v[pl.ds(base, 16)]
                acc1 = acc1 + ak * h_v[pl.ds(base + 16, 16)]
                acc2 = acc2 + ak * h_v[pl.ds(base + 32, 16)]
                acc3 = acc3 + ak * h_v[pl.ds(base + 48, 16)]
        r = (jnp.maximum(acc0, 0.0) * w0
             + jnp.maximum(acc1, 0.0) * w1
             + jnp.maximum(acc2, 0.0) * w2
             + jnp.maximum(acc3, 0.0) * w3)
        y = jnp.sum(r)
        plsc.store_scatter(y_v, [nvec], jnp.full((16,), y, jnp.float32),
                           mask=lane0)

    @plsc.parallel_loop(0, NPW, unroll=2)
    def _(n):
        one_node(n)
    pltpu.sync_copy(y_v, y_hbm.at[pl.ds(b * N + nbase, NPW)])


@functools.cache
def _get_sc_edge():
    mesh = plsc.VectorSubcoreMesh(core_axis_name="c", subcore_axis_name="s")
    return pl.kernel(
        _sc_edge_body,
        out_type=jax.ShapeDtypeStruct((B * N,), jnp.float32),
        mesh=mesh,
        compiler_params=pltpu.CompilerParams(needs_layout_passes=False),
        scratch_types=[
            pltpu.VMEM((N * E,), jnp.float32),
            pltpu.VMEM((N,), jnp.float32),
            pltpu.VMEM((NPW,), jnp.float32),
            pltpu.VMEM((NPW * K,), jnp.int32),
            pltpu.VMEM((E,), jnp.float32),
            pltpu.VMEM((NPW,), jnp.float32),
        ],
    )


def kernel(x, emb_table, W, b_fe, a_l, a_r, W_out, b_out):
    h, sl, sr = _feat_call(x, W, b_fe.reshape(1, E),
                           a_l.reshape(1, 2 * E), a_r.reshape(1, 2 * E),
                           emb_table)
    idx_t = _topk_call(emb_table, emb_table)
    idx = jnp.transpose(idx_t, (0, 2, 1)).reshape(N, K)
    y = _get_sc_edge()(h.reshape(-1), sl.reshape(-1), sr.reshape(-1),
                       idx.reshape(-1), W_out.reshape(-1))
    return y.reshape(B, N) + b_out[0]


# speedup vs baseline: 164.1187x; 1.1849x over previous
import functools

import jax
import jax.numpy as jnp
from jax import lax
from jax.experimental import pallas as pl
from jax.experimental.pallas import tpu as pltpu
from jax.experimental.pallas import tpu_sc as plsc

N = 1024
B = 16
F = 128
E = 64
K = 32
RT = 128
NW = 32
NPW = (B * N) // NW


def _feat_body(x_ref, w_ref, b_ref, al_ref, ar_ref, emb_ref,
               h_ref, sl_ref, sr_ref):
    xb = x_ref[0]
    h = jnp.dot(xb, w_ref[...], preferred_element_type=jnp.float32)
    h = h + b_ref[0]
    h_ref[0] = h
    emb = emb_ref[...]
    al = al_ref[0]
    ar = ar_ref[0]
    sl = (jnp.sum(h * al[None, :E], axis=1)
          + jnp.sum(emb * al[None, E:], axis=1))
    sr = (jnp.sum(h * ar[None, :E], axis=1)
          + jnp.sum(emb * ar[None, E:], axis=1))
    sl_ref[0] = sl.reshape(8, N // 8)
    sr_ref[0] = sr.reshape(8, N // 8)


_feat_call = pl.pallas_call(
    _feat_body,
    grid=(B,),
    in_specs=[
        pl.BlockSpec((1, N, F), lambda i: (i, 0, 0)),
        pl.BlockSpec((F, E), lambda i: (0, 0)),
        pl.BlockSpec((1, E), lambda i: (0, 0)),
        pl.BlockSpec((1, 2 * E), lambda i: (0, 0)),
        pl.BlockSpec((1, 2 * E), lambda i: (0, 0)),
        pl.BlockSpec((N, E), lambda i: (0, 0)),
    ],
    out_specs=[
        pl.BlockSpec((1, N, E), lambda i: (i, 0, 0)),
        pl.BlockSpec((1, 8, N // 8), lambda i: (i, 0, 0)),
        pl.BlockSpec((1, 8, N // 8), lambda i: (i, 0, 0)),
    ],
    out_shape=[
        jax.ShapeDtypeStruct((B, N, E), jnp.float32),
        jax.ShapeDtypeStruct((B, 8, N // 8), jnp.float32),
        jax.ShapeDtypeStruct((B, 8, N // 8), jnp.float32),
    ],
)


def _topk_body(emb_ref, idx_ref):
    e = emb_ref[...]
    nf = jnp.sqrt(jnp.sum(e * e, axis=1))
    dot = lax.dot_general(e, e, (((1,), (1,)), ((), ())),
                          preferred_element_type=jnp.float32)
    s0 = dot / nf[:, None]
    rowio = lax.broadcasted_iota(jnp.int32, (N, N), 0)
    kio = lax.broadcasted_iota(jnp.int32, (K, N), 0)

    def body(k, carry):
        s, idxs = carry
        m = jnp.max(s, axis=0, keepdims=True)
        am = jnp.min(jnp.where(s >= m, rowio, N), axis=0, keepdims=True)
        idxs = jnp.where(kio == k, am, idxs)
        s = jnp.where(rowio == am, -jnp.inf, s)
        return s, idxs

    _, idxs = lax.fori_loop(0, K, body,
                            (s0, jnp.zeros((K, N), jnp.int32)))
    idx_ref[...] = idxs


_topk_call = pl.pallas_call(
    _topk_body,
    out_shape=jax.ShapeDtypeStruct((K, N), jnp.int32),
)


def _sc_edge_body(h_hbm, sl_hbm, sr_hbm, idx_hbm, wout_hbm, y_hbm,
                  h_v, sr_v, sl_v, idx_v, wout_v, y_v):
    cid = lax.axis_index("c")
    sid = lax.axis_index("s")
    w = sid * 2 + cid
    b = w // 2
    half = w % 2
    nbase = half * NPW

    pltpu.sync_copy(h_hbm.at[pl.ds(b * N * E, N * E)], h_v)
    pltpu.sync_copy(sr_hbm.at[pl.ds(b * N, N)], sr_v)
    pltpu.sync_copy(sl_hbm.at[pl.ds(b * N + nbase, NPW)], sl_v)
    pltpu.sync_copy(idx_hbm.at[pl.ds(nbase * K, NPW * K)], idx_v)
    pltpu.sync_copy(wout_hbm, wout_v)

    lanei = lax.iota(jnp.int32, 16)
    lane0 = lanei == 0
    w0 = wout_v[0:16]
    w1 = wout_v[16:32]
    w2 = wout_v[32:48]
    w3 = wout_v[48:64]

    def one_node(n):
        iv0 = idx_v[pl.ds(n * K, 16)]
        iv1 = idx_v[pl.ds(n * K + 16, 16)]
        nvec = jnp.full((16,), n, jnp.int32)
        s0 = plsc.load_gather(sr_v, [iv0])
        s1 = plsc.load_gather(sr_v, [iv1])
        slv = plsc.load_gather(sl_v, [nvec])
        l0 = slv + s0
        l1 = slv + s1
        l0 = jnp.where(l0 >= 0, l0, 0.2 * l0)
        l1 = jnp.where(l1 >= 0, l1, 0.2 * l1)
        e0 = jnp.exp(l0)
        e1 = jnp.exp(l1)
        denom = jnp.full((16,), jnp.sum(e0) + jnp.sum(e1), jnp.float32)
        a0 = e0 / denom
        a1 = e1 / denom
        ad0 = iv0 * E
        ad1 = iv1 * E
        acc0 = jnp.zeros((16,), jnp.float32)
        acc1 = jnp.zeros((16,), jnp.float32)
        acc2 = jnp.zeros((16,), jnp.float32)
        acc3 = jnp.zeros((16,), jnp.float32)
        for av, adv in ((a0, ad0), (a1, ad1)):
            for k in range(16):
                ak = av[k]
                base = adv[k]
                acc0 = acc0 + ak * h_v[pl.ds(base, 16)]
                acc1 = acc1 + ak * h_v[pl.ds(base + 16, 16)]
                acc2 = acc2 + ak * h_v[pl.ds(base + 32, 16)]
                acc3 = acc3 + ak * h_v[pl.ds(base + 48, 16)]
        r = (jnp.maximum(acc0, 0.0) * w0
             + jnp.maximum(acc1, 0.0) * w1
             + jnp.maximum(acc2, 0.0) * w2
             + jnp.maximum(acc3, 0.0) * w3)
        y = jnp.sum(r)
        plsc.store_scatter(y_v, [nvec], jnp.full((16,), y, jnp.float32),
                           mask=lane0)

    @plsc.parallel_loop(0, NPW, unroll=2)
    def _(n):
        one_node(n)
    pltpu.sync_copy(y_v, y_hbm.at[pl.ds(b * N + nbase, NPW)])


@functools.cache
def _get_sc_edge():
    mesh = plsc.VectorSubcoreMesh(core_axis_name="c", subcore_axis_name="s")
    return pl.kernel(
        _sc_edge_body,
        out_type=jax.ShapeDtypeStruct((B * N,), jnp.float32),
        mesh=mesh,
        compiler_params=pltpu.CompilerParams(needs_layout_passes=False),
        scratch_types=[
            pltpu.VMEM((N * E,), jnp.float32),
            pltpu.VMEM((N,), jnp.float32),
            pltpu.VMEM((NPW,), jnp.float32),
            pltpu.VMEM((NPW * K,), jnp.int32),
            pltpu.VMEM((E,), jnp.float32),
            pltpu.VMEM((NPW,), jnp.float32),
        ],
    )


def kernel(x, emb_table, W, b_fe, a_l, a_r, W_out, b_out):
    h, sl, sr = _feat_call(x, W, b_fe.reshape(1, E),
                           a_l.reshape(1, 2 * E), a_r.reshape(1, 2 * E),
                           emb_table)
    idx_t = _topk_call(emb_table)
    idx = idx_t.T
    y = _get_sc_edge()(h.reshape(-1), sl.reshape(-1), sr.reshape(-1),
                       idx.reshape(-1), W_out.reshape(-1))
    return y.reshape(B, N) + b_out[0]
